# baseline (speedup 1.0000x reference)
import jax
import jax.numpy as jnp
from jax import lax
from jax.experimental import pallas as pl
from jax.experimental.pallas import tpu as pltpu
from jax.experimental.pallas import tpu_sc as plsc

_FIELD = 26
_K = 16
_NPAIR = _FIELD * (_FIELD - 1) // 2


def _tc_body(e_ref, w_ref, b_ref, r_ref, out_ref, v_ref):
    for f in range(_FIELD):
        vf = lax.dot_general(w_ref[...], e_ref[f], (((1,), (1,)), ((), ())),
                             preferred_element_type=jnp.float32)
        v_ref[f] = vf + b_ref[...]
    off = 0
    for r in range(_FIELD - 1):
        n = _FIELD - 1 - r
        vr = v_ref[r] * r_ref[...]
        rest = v_ref[pl.ds(r + 1, n)]
        out_ref[pl.ds(off, n)] = jnp.sum(rest * vr[None, :, :], axis=1)
        off += n


def _tc_pairs(e3, W, b2, r2, bb):
    F, B, K = e3.shape
    return pl.pallas_call(
        _tc_body,
        grid=(B // bb,),
        in_specs=[
            pl.BlockSpec((F, bb, K), lambda i: (0, i, 0)),
            pl.BlockSpec((K, K), lambda i: (0, 0)),
            pl.BlockSpec((K, 1), lambda i: (0, 0)),
            pl.BlockSpec((K, 1), lambda i: (0, 0)),
        ],
        out_specs=pl.BlockSpec((_NPAIR, bb), lambda i: (0, i)),
        out_shape=jax.ShapeDtypeStruct((_NPAIR, B), jnp.float32),
        scratch_shapes=[pltpu.VMEM((F, K, bb), jnp.float32)],
    )(e3, W, b2, r2)


_RCH = 512
_PK = _RCH // 8
_NBUF = 3


def _sc_repack(tT, tail_tT):
    V = tT.shape[1]
    npk = V // 8
    info = plsc.get_sparse_core_info()
    nc, ns = info.num_cores, info.num_subcores
    nw = nc * ns
    nch = V // _RCH
    tail = V - nch * _RCH
    cpw = -(-nch // nw)
    cpw += (-cpw) % _NBUF
    mesh = plsc.VectorSubcoreMesh(core_axis_name="c", subcore_axis_name="s")

    def body(t_hbm, tail_hbm, out_hbm, bufs, pks, gsems, osems):
        wid = lax.axis_index("s") * nc + lax.axis_index("c")
        iota = lax.iota(jnp.int32, _K)
        rowc = iota >> 3
        lanec = [(iota & 7) * _K + k for k in range(_K)]

        def i0_of(t):
            ci = jnp.minimum(t * nw + wid, nch - 1)
            return pl.multiple_of(ci * _RCH, _RCH)

        def fire(t, j):
            pltpu.async_copy(t_hbm.at[:, pl.ds(i0_of(t), _RCH)], bufs.at[j],
                             gsems.at[j])

        def wait_in(j):
            pltpu.make_async_copy(t_hbm.at[:, pl.ds(0, _RCH)], bufs.at[j],
                                  gsems.at[j]).wait()

        def wait_out(j):
            pltpu.make_async_copy(pks.at[j], out_hbm.at[pl.ds(0, _PK), :],
                                  osems.at[j]).wait()

        def transpose_into(j, n):
            for e0 in range(n // _K):
                dst = pks.at[j, pl.ds(e0 * 2, 2), :]
                for k in range(_K):
                    v = bufs[j, k, pl.ds(e0 * _K, _K)]
                    plsc.store_scatter(dst, [rowc, lanec[k]], v)

        def compact_write(t, j):
            transpose_into(j, _RCH)
            pltpu.async_copy(
                pks.at[j], out_hbm.at[pl.ds(pl.multiple_of(i0_of(t) // 8, _PK),
                                            _PK), :],
                osems.at[j])

        for j in range(_NBUF):
            fire(j, j)

        def step(i, carry):
            for j in range(_NBUF):
                wait_in(j)

                @pl.when(i > 0)
                def _():
                    wait_out(j)

                compact_write(_NBUF * i + j, j)
                fire(jnp.minimum(_NBUF * i + _NBUF + j, cpw - 1), j)
            return carry

        lax.fori_loop(0, cpw // _NBUF, step, 0)
        for j in range(_NBUF):
            wait_in(j)
            wait_out(j)

        if tail:
            @pl.when(wid == 0)
            def _():
                pltpu.sync_copy(tail_hbm, bufs.at[0, :, pl.ds(0, 128)])
                transpose_into(0, 128)
                pltpu.sync_copy(pks.at[0, pl.ds(0, 16), :],
                                out_hbm.at[pl.ds(npk - 16, 16), :])

    f = pl.kernel(
        body,
        out_type=jax.ShapeDtypeStruct((npk, 128), jnp.float32),
        mesh=mesh,
        compiler_params=pltpu.CompilerParams(needs_layout_passes=False),
        scratch_types=[
            pltpu.VMEM((_NBUF, _K, _RCH), jnp.float32),
            pltpu.VMEM((_NBUF, _PK, 128), jnp.float32),
            pltpu.SemaphoreType.DMA((_NBUF,)),
            pltpu.SemaphoreType.DMA((_NBUF,)),
        ],
    )
    return f(tT, tail_tT)


_CH = 16
_DEPTH = 4


def _sc_gather_fm(xT, tp):
    B = xT.shape[1]
    info = plsc.get_sparse_core_info()
    nc, ns = info.num_cores, info.num_subcores
    nw = nc * ns
    bw = B // nw
    gpf = bw // _CH
    nch = gpf * _FIELD
    mesh = plsc.VectorSubcoreMesh(core_axis_name="c", subcore_axis_name="s")

    def body(x_hbm, t_hbm, out_hbm, xv, tiles, rows, gsems, osems):
        wid = lax.axis_index("s") * nc + lax.axis_index("c")
        b0 = pl.multiple_of(wid * bw, bw)
        pltpu.sync_copy(x_hbm.at[:, pl.ds(b0, bw)], xv)
        iota = lax.iota(jnp.int32, _CH)

        def fire(q, j):
            f = q // gpf
            g = q - f * gpf
            raw = xv[f, pl.ds(g * _CH, _CH)]
            pltpu.async_copy(t_hbm.at[raw >> 3], tiles.at[j], gsems.at[j])
            return raw & 7

        def wait_gather(j):
            pltpu.make_async_copy(
                t_hbm.at[iota], tiles.at[j], gsems.at[j]).wait()

        def wait_out(j):
            pltpu.make_async_copy(
                rows.at[j], out_hbm.at[0, pl.ds(0, _CH), :], osems.at[j]).wait()

        def extract_write(q, j, sub):
            f = q // gpf
            g = q - f * gpf
            for k in range(_K):
                val = plsc.load_gather(tiles.at[j], [iota, sub * _K + k])
                plsc.store_scatter(
                    rows.at[j], [iota, jnp.full((_CH,), k, jnp.int32)], val)
            pltpu.async_copy(
                rows.at[j],
                out_hbm.at[f, pl.ds(pl.multiple_of(b0 + g * _CH, _CH), _CH), :],
                osems.at[j])

        subs0 = tuple(fire(q, q) for q in range(_DEPTH))

        def step(i, subs):
            new_subs = []
            for j in range(_DEPTH):
                q = i * _DEPTH + j
                wait_gather(j)

                @pl.when(i > 0)
                def _():
                    wait_out(j)

                extract_write(q, j, subs[j])
                nq = jnp.minimum(q + _DEPTH, nch - 1)
                new_subs.append(fire(nq, j))
            return tuple(new_subs)

        _ = lax.fori_loop(0, nch // _DEPTH, step, subs0)
        for j in range(_DEPTH):
            wait_gather(j)
            wait_out(j)

    f = pl.kernel(
        body,
        out_type=jax.ShapeDtypeStruct((_FIELD, B, _K), jnp.float32),
        mesh=mesh,
        compiler_params=pltpu.CompilerParams(needs_layout_passes=False),
        scratch_types=[
            pltpu.VMEM((_FIELD, bw), jnp.int32),
            pltpu.VMEM((_DEPTH, _CH, 128), jnp.float32),
            pltpu.VMEM((_DEPTH, _CH, _K), jnp.float32),
            pltpu.SemaphoreType.DMA((_DEPTH,)),
            pltpu.SemaphoreType.DMA((_DEPTH,)),
        ],
    )
    return f(xT, tp)


def kernel(x, table, W, b, fi_rank):
    B, F = x.shape
    tT = table.T
    tp = _sc_repack(tT, tT[:, -128:])
    e3 = _sc_gather_fm(x.T, tp)
    outT = _tc_pairs(e3, W, b.reshape(_K, 1), fi_rank.reshape(_K, 1), 512)
    return outT.T


# speedup vs baseline: 1.0959x; 1.0959x over previous
import jax
import jax.numpy as jnp
from jax import lax
from jax.experimental import pallas as pl
from jax.experimental.pallas import tpu as pltpu
from jax.experimental.pallas import tpu_sc as plsc

_FIELD = 26
_K = 16
_NPAIR = _FIELD * (_FIELD - 1) // 2


def _tc_body(e_ref, w_ref, b_ref, r_ref, out_ref, v_ref):
    for f in range(_FIELD):
        vf = lax.dot_general(w_ref[...], e_ref[f], (((1,), (1,)), ((), ())),
                             preferred_element_type=jnp.float32)
        v_ref[f] = vf + b_ref[...]
    off = 0
    for r in range(_FIELD - 1):
        n = _FIELD - 1 - r
        vr = v_ref[r] * r_ref[...]
        rest = v_ref[pl.ds(r + 1, n)]
        out_ref[pl.ds(off, n)] = jnp.sum(rest * vr[None, :, :], axis=1)
        off += n


def _tc_pairs(e3, W, b2, r2, bb):
    F, B, K = e3.shape
    return pl.pallas_call(
        _tc_body,
        grid=(B // bb,),
        in_specs=[
            pl.BlockSpec((F, bb, K), lambda i: (0, i, 0)),
            pl.BlockSpec((K, K), lambda i: (0, 0)),
            pl.BlockSpec((K, 1), lambda i: (0, 0)),
            pl.BlockSpec((K, 1), lambda i: (0, 0)),
        ],
        out_specs=pl.BlockSpec((_NPAIR, bb), lambda i: (0, i)),
        out_shape=jax.ShapeDtypeStruct((_NPAIR, B), jnp.float32),
        scratch_shapes=[pltpu.VMEM((F, K, bb), jnp.float32)],
    )(e3, W, b2, r2)


_RCH = 256
_PK = _RCH // 8
_NBUF = 3


def _sc_repack(tT, tail_tT):
    V = tT.shape[1]
    npk = V // 8
    info = plsc.get_sparse_core_info()
    nc, ns = info.num_cores, info.num_subcores
    nw = nc * ns
    nch = V // _RCH
    tail = V - nch * _RCH
    cpw = -(-nch // nw)
    cpw += (-cpw) % _NBUF
    mesh = plsc.VectorSubcoreMesh(core_axis_name="c", subcore_axis_name="s")

    def body(t_hbm, tail_hbm, out_hbm, bufs, pks, gsems, osems):
        wid = lax.axis_index("s") * nc + lax.axis_index("c")
        iota = lax.iota(jnp.int32, _K)
        rowc = iota >> 3
        lanec = [(iota & 7) * _K + k for k in range(_K)]

        def i0_of(t):
            ci = jnp.minimum(t * nw + wid, nch - 1)
            return pl.multiple_of(ci * _RCH, _RCH)

        def fire(t, j):
            pltpu.async_copy(t_hbm.at[:, pl.ds(i0_of(t), _RCH)], bufs.at[j],
                             gsems.at[j])

        def wait_in(j):
            pltpu.make_async_copy(t_hbm.at[:, pl.ds(0, _RCH)], bufs.at[j],
                                  gsems.at[j]).wait()

        def wait_out(j):
            pltpu.make_async_copy(pks.at[j], out_hbm.at[pl.ds(0, _PK), :],
                                  osems.at[j]).wait()

        def transpose_into(j, n):
            for e0 in range(n // _K):
                dst = pks.at[j, pl.ds(e0 * 2, 2), :]
                for k in range(_K):
                    v = bufs[j, k, pl.ds(e0 * _K, _K)]
                    plsc.store_scatter(dst, [rowc, lanec[k]], v)

        def compact_write(t, j):
            transpose_into(j, _RCH)
            pltpu.async_copy(
                pks.at[j], out_hbm.at[pl.ds(pl.multiple_of(i0_of(t) // 8, _PK),
                                            _PK), :],
                osems.at[j])

        for j in range(_NBUF):
            fire(j, j)

        def step(i, carry):
            for j in range(_NBUF):
                wait_in(j)

                @pl.when(i > 0)
                def _():
                    wait_out(j)

                compact_write(_NBUF * i + j, j)
                fire(jnp.minimum(_NBUF * i + _NBUF + j, cpw - 1), j)
            return carry

        lax.fori_loop(0, cpw // _NBUF, step, 0)
        for j in range(_NBUF):
            wait_in(j)
            wait_out(j)

        if tail:
            @pl.when(wid == 0)
            def _():
                pltpu.sync_copy(tail_hbm, bufs.at[0, :, pl.ds(0, 128)])
                transpose_into(0, 128)
                pltpu.sync_copy(pks.at[0, pl.ds(0, 16), :],
                                out_hbm.at[pl.ds(npk - 16, 16), :])

    f = pl.kernel(
        body,
        out_type=jax.ShapeDtypeStruct((npk, 128), jnp.float32),
        mesh=mesh,
        compiler_params=pltpu.CompilerParams(needs_layout_passes=False),
        scratch_types=[
            pltpu.VMEM((_NBUF, _K, _RCH), jnp.float32),
            pltpu.VMEM((_NBUF, _PK, 128), jnp.float32),
            pltpu.SemaphoreType.DMA((_NBUF,)),
            pltpu.SemaphoreType.DMA((_NBUF,)),
        ],
    )
    return f(tT, tail_tT)


_CH = 16
_DEPTH = 4


def _sc_gather_fm(xT, tp):
    B = xT.shape[1]
    info = plsc.get_sparse_core_info()
    nc, ns = info.num_cores, info.num_subcores
    nw = nc * ns
    bw = B // nw
    gpf = bw // _CH
    nch = gpf * _FIELD
    mesh = plsc.VectorSubcoreMesh(core_axis_name="c", subcore_axis_name="s")

    def body(x_hbm, t_hbm, out_hbm, xv, tiles, rows, gsems, osems):
        wid = lax.axis_index("s") * nc + lax.axis_index("c")
        b0 = pl.multiple_of(wid * bw, bw)
        pltpu.sync_copy(x_hbm.at[:, pl.ds(b0, bw)], xv)
        iota = lax.iota(jnp.int32, _CH)

        def fire(q, j):
            f = q // gpf
            g = q - f * gpf
            raw = xv[f, pl.ds(g * _CH, _CH)]
            pltpu.async_copy(t_hbm.at[raw >> 3], tiles.at[j], gsems.at[j])
            return raw & 7

        def wait_gather(j):
            pltpu.make_async_copy(
                t_hbm.at[iota], tiles.at[j], gsems.at[j]).wait()

        def wait_out(j):
            pltpu.make_async_copy(
                rows.at[j], out_hbm.at[0, pl.ds(0, _CH), :], osems.at[j]).wait()

        def extract_write(q, j, sub):
            f = q // gpf
            g = q - f * gpf
            for k in range(_K):
                val = plsc.load_gather(tiles.at[j], [iota, sub * _K + k])
                plsc.store_scatter(
                    rows.at[j], [iota, jnp.full((_CH,), k, jnp.int32)], val)
            pltpu.async_copy(
                rows.at[j],
                out_hbm.at[f, pl.ds(pl.multiple_of(b0 + g * _CH, _CH), _CH), :],
                osems.at[j])

        subs0 = tuple(fire(q, q) for q in range(_DEPTH))

        def step(i, subs):
            new_subs = []
            for j in range(_DEPTH):
                q = i * _DEPTH + j
                wait_gather(j)

                @pl.when(i > 0)
                def _():
                    wait_out(j)

                extract_write(q, j, subs[j])
                nq = jnp.minimum(q + _DEPTH, nch - 1)
                new_subs.append(fire(nq, j))
            return tuple(new_subs)

        _ = lax.fori_loop(0, nch // _DEPTH, step, subs0)
        for j in range(_DEPTH):
            wait_gather(j)
            wait_out(j)

    f = pl.kernel(
        body,
        out_type=jax.ShapeDtypeStruct((_FIELD, B, _K), jnp.float32),
        mesh=mesh,
        compiler_params=pltpu.CompilerParams(needs_layout_passes=False),
        scratch_types=[
            pltpu.VMEM((_FIELD, bw), jnp.int32),
            pltpu.VMEM((_DEPTH, _CH, 128), jnp.float32),
            pltpu.VMEM((_DEPTH, _CH, _K), jnp.float32),
            pltpu.SemaphoreType.DMA((_DEPTH,)),
            pltpu.SemaphoreType.DMA((_DEPTH,)),
        ],
    )
    return f(xT, tp)


def kernel(x, table, W, b, fi_rank):
    B, F = x.shape
    tT = table.T
    tp = _sc_repack(tT, tT[:, -128:])
    e3 = _sc_gather_fm(x.T, tp)
    outT = _tc_pairs(e3, W, b.reshape(_K, 1), fi_rank.reshape(_K, 1), 512)
    return outT.T


# speedup vs baseline: 1.2641x; 1.1534x over previous
import jax
import jax.numpy as jnp
from jax import lax
from jax.experimental import pallas as pl
from jax.experimental.pallas import tpu as pltpu
from jax.experimental.pallas import tpu_sc as plsc

_FIELD = 26
_K = 16
_NPAIR = _FIELD * (_FIELD - 1) // 2


def _tc_body(e_ref, w_ref, b_ref, r_ref, out_ref, v_ref):
    for f in range(_FIELD):
        vf = lax.dot_general(w_ref[...], e_ref[f], (((1,), (0,)), ((), ())),
                             preferred_element_type=jnp.float32)
        v_ref[f] = vf + b_ref[...]
    off = 0
    for r in range(_FIELD - 1):
        n = _FIELD - 1 - r
        vr = v_ref[r] * r_ref[...]
        rest = v_ref[pl.ds(r + 1, n)]
        out_ref[pl.ds(off, n)] = jnp.sum(rest * vr[None, :, :], axis=1)
        off += n


def _tc_pairs(e3, W, b2, r2, bb):
    F, K, B = e3.shape
    return pl.pallas_call(
        _tc_body,
        grid=(B // bb,),
        in_specs=[
            pl.BlockSpec((F, K, bb), lambda i: (0, 0, i)),
            pl.BlockSpec((K, K), lambda i: (0, 0)),
            pl.BlockSpec((K, 1), lambda i: (0, 0)),
            pl.BlockSpec((K, 1), lambda i: (0, 0)),
        ],
        out_specs=pl.BlockSpec((_NPAIR, bb), lambda i: (0, i)),
        out_shape=jax.ShapeDtypeStruct((_NPAIR, B), jnp.float32),
        scratch_shapes=[pltpu.VMEM((F, K, bb), jnp.float32)],
    )(e3, W, b2, r2)


_RCH = 256
_PK = _RCH // 8
_NBUF = 3


def _sc_repack(tT, tail_tT):
    V = tT.shape[1]
    npk = V // 8
    info = plsc.get_sparse_core_info()
    nc, ns = info.num_cores, info.num_subcores
    nw = nc * ns
    nch = V // _RCH
    tail = V - nch * _RCH
    cpw = -(-nch // nw)
    cpw += (-cpw) % _NBUF
    mesh = plsc.VectorSubcoreMesh(core_axis_name="c", subcore_axis_name="s")

    def body(t_hbm, tail_hbm, out_hbm, bufs, pks, gsems, osems):
        wid = lax.axis_index("s") * nc + lax.axis_index("c")
        iota = lax.iota(jnp.int32, _K)
        rowc = iota >> 3
        lanec = [(iota & 7) * _K + k for k in range(_K)]

        def i0_of(t):
            ci = jnp.minimum(t * nw + wid, nch - 1)
            return pl.multiple_of(ci * _RCH, _RCH)

        def fire(t, j):
            pltpu.async_copy(t_hbm.at[:, pl.ds(i0_of(t), _RCH)], bufs.at[j],
                             gsems.at[j])

        def wait_in(j):
            pltpu.make_async_copy(t_hbm.at[:, pl.ds(0, _RCH)], bufs.at[j],
                                  gsems.at[j]).wait()

        def wait_out(j):
            pltpu.make_async_copy(pks.at[j], out_hbm.at[pl.ds(0, _PK), :],
                                  osems.at[j]).wait()

        def transpose_into(j, n):
            for e0 in range(n // _K):
                dst = pks.at[j, pl.ds(e0 * 2, 2), :]
                for k in range(_K):
                    v = bufs[j, k, pl.ds(e0 * _K, _K)]
                    plsc.store_scatter(dst, [rowc, lanec[k]], v)

        def compact_write(t, j):
            transpose_into(j, _RCH)
            pltpu.async_copy(
                pks.at[j], out_hbm.at[pl.ds(pl.multiple_of(i0_of(t) // 8, _PK),
                                            _PK), :],
                osems.at[j])

        for j in range(_NBUF):
            fire(j, j)

        def step(i, carry):
            for j in range(_NBUF):
                wait_in(j)

                @pl.when(i > 0)
                def _():
                    wait_out(j)

                compact_write(_NBUF * i + j, j)
                fire(jnp.minimum(_NBUF * i + _NBUF + j, cpw - 1), j)
            return carry

        lax.fori_loop(0, cpw // _NBUF, step, 0)
        for j in range(_NBUF):
            wait_in(j)
            wait_out(j)

        if tail:
            @pl.when(wid == 0)
            def _():
                pltpu.sync_copy(tail_hbm, bufs.at[0, :, pl.ds(0, 128)])
                transpose_into(0, 128)
                pltpu.sync_copy(pks.at[0, pl.ds(0, 16), :],
                                out_hbm.at[pl.ds(npk - 16, 16), :])

    f = pl.kernel(
        body,
        out_type=jax.ShapeDtypeStruct((npk, 128), jnp.float32),
        mesh=mesh,
        compiler_params=pltpu.CompilerParams(needs_layout_passes=False),
        scratch_types=[
            pltpu.VMEM((_NBUF, _K, _RCH), jnp.float32),
            pltpu.VMEM((_NBUF, _PK, 128), jnp.float32),
            pltpu.SemaphoreType.DMA((_NBUF,)),
            pltpu.SemaphoreType.DMA((_NBUF,)),
        ],
    )
    return f(tT, tail_tT)


_CH = 16


def _sc_gather_fm(xT, tp):
    B = xT.shape[1]
    info = plsc.get_sparse_core_info()
    nc, ns = info.num_cores, info.num_subcores
    nw = nc * ns
    bw = B // nw
    gpf = bw // _CH
    mesh = plsc.VectorSubcoreMesh(core_axis_name="c", subcore_axis_name="s")

    def body(x_hbm, t_hbm, out_hbm, xv, tiles, rows, gsems, osems):
        wid = lax.axis_index("s") * nc + lax.axis_index("c")
        b0 = pl.multiple_of(wid * bw, bw)
        pltpu.sync_copy(x_hbm.at[:, pl.ds(b0, bw)], xv)
        iota = lax.iota(jnp.int32, _CH)

        def fire8(f, p):
            subs = []
            for c in range(gpf):
                raw = xv[f, pl.ds(c * _CH, _CH)]
                pltpu.async_copy(t_hbm.at[raw >> 3], tiles.at[p, c],
                                 gsems.at[p])
                subs.append((raw & 7) * _K)
            return tuple(subs)

        def drain8(p):
            for c in range(gpf):
                pltpu.make_async_copy(t_hbm.at[iota], tiles.at[p, c],
                                      gsems.at[p]).wait()

        def wait_out(p):
            pltpu.make_async_copy(rows.at[p], out_hbm.at[0, :, pl.ds(0, bw)],
                                  osems.at[p]).wait()

        def extract_write(f, p, subs):
            for c in range(gpf):
                for k in range(_K):
                    val = plsc.load_gather(tiles.at[p, c], [iota, subs[c] + k])
                    rows[p, k, pl.ds(c * _CH, _CH)] = val
            pltpu.async_copy(rows.at[p], out_hbm.at[f, :, pl.ds(b0, bw)],
                             osems.at[p])

        carry0 = fire8(0, 0) + fire8(1, 1)

        def step(i, carry):
            subsA, subsB = carry[:gpf], carry[gpf:]
            f0 = 2 * i
            new = []
            for p, subs, f in ((0, subsA, f0), (1, subsB, f0 + 1)):
                drain8(p)

                @pl.when(i > 0)
                def _():
                    wait_out(p)

                extract_write(f, p, subs)
                new.extend(fire8(jnp.minimum(f + 2, _FIELD - 1), p))
            return tuple(new)

        _ = lax.fori_loop(0, _FIELD // 2, step, carry0)
        for p in range(2):
            drain8(p)
            wait_out(p)

    f = pl.kernel(
        body,
        out_type=jax.ShapeDtypeStruct((_FIELD, _K, B), jnp.float32),
        mesh=mesh,
        compiler_params=pltpu.CompilerParams(needs_layout_passes=False),
        scratch_types=[
            pltpu.VMEM((_FIELD, bw), jnp.int32),
            pltpu.VMEM((2, gpf, _CH, 128), jnp.float32),
            pltpu.VMEM((2, _K, bw), jnp.float32),
            pltpu.SemaphoreType.DMA((2,)),
            pltpu.SemaphoreType.DMA((2,)),
        ],
    )
    return f(xT, tp)


def kernel(x, table, W, b, fi_rank):
    B, F = x.shape
    tT = table.T
    tp = _sc_repack(tT, tT[:, -128:])
    e3 = _sc_gather_fm(x.T, tp)
    outT = _tc_pairs(e3, W, b.reshape(_K, 1), fi_rank.reshape(_K, 1), 512)
    return outT.T


# speedup vs baseline: 1.3053x; 1.0326x over previous
import jax
import jax.numpy as jnp
from jax import lax
from jax.experimental import pallas as pl
from jax.experimental.pallas import tpu as pltpu
from jax.experimental.pallas import tpu_sc as plsc

_FIELD = 26
_K = 16
_NPAIR = _FIELD * (_FIELD - 1) // 2


def _tc_body(e_ref, w_ref, b_ref, r_ref, out_ref, v_ref):
    for f in range(_FIELD):
        vf = lax.dot_general(w_ref[...], e_ref[f], (((1,), (0,)), ((), ())),
                             preferred_element_type=jnp.float32)
        v_ref[f] = vf + b_ref[...]
    off = 0
    for r in range(_FIELD - 1):
        n = _FIELD - 1 - r
        vr = v_ref[r] * r_ref[...]
        rest = v_ref[pl.ds(r + 1, n)]
        out_ref[pl.ds(off, n)] = jnp.sum(rest * vr[None, :, :], axis=1)
        off += n


def _tc_pairs(e3, W, b2, r2, bb):
    F, K, B = e3.shape
    return pl.pallas_call(
        _tc_body,
        grid=(B // bb,),
        in_specs=[
            pl.BlockSpec((F, K, bb), lambda i: (0, 0, i)),
            pl.BlockSpec((K, K), lambda i: (0, 0)),
            pl.BlockSpec((K, 1), lambda i: (0, 0)),
            pl.BlockSpec((K, 1), lambda i: (0, 0)),
        ],
        out_specs=pl.BlockSpec((_NPAIR, bb), lambda i: (0, i)),
        out_shape=jax.ShapeDtypeStruct((_NPAIR, B), jnp.float32),
        scratch_shapes=[pltpu.VMEM((F, K, bb), jnp.float32)],
    )(e3, W, b2, r2)


_RCH = 128
_PK = _RCH // 8
_NBUF = 3


def _sc_repack(tT, tail_tT):
    V = tT.shape[1]
    npk = V // 8
    info = plsc.get_sparse_core_info()
    nc, ns = info.num_cores, info.num_subcores
    nw = nc * ns
    nch = V // _RCH
    tail = V - nch * _RCH
    cpw = -(-nch // nw)
    cpw += (-cpw) % _NBUF
    mesh = plsc.VectorSubcoreMesh(core_axis_name="c", subcore_axis_name="s")

    def body(t_hbm, tail_hbm, out_hbm, bufs, pks, gsems, osems):
        wid = lax.axis_index("s") * nc + lax.axis_index("c")
        iota = lax.iota(jnp.int32, _K)
        rowc = iota >> 3
        lanec = [(iota & 7) * _K + k for k in range(_K)]

        def i0_of(t):
            ci = jnp.minimum(t * nw + wid, nch - 1)
            return pl.multiple_of(ci * _RCH, _RCH)

        def fire(t, j):
            pltpu.async_copy(t_hbm.at[:, pl.ds(i0_of(t), _RCH)], bufs.at[j],
                             gsems.at[j])

        def wait_in(j):
            pltpu.make_async_copy(t_hbm.at[:, pl.ds(0, _RCH)], bufs.at[j],
                                  gsems.at[j]).wait()

        def wait_out(j):
            pltpu.make_async_copy(pks.at[j], out_hbm.at[pl.ds(0, _PK), :],
                                  osems.at[j]).wait()

        def transpose_into(j, n):
            for e0 in range(n // _K):
                dst = pks.at[j, pl.ds(e0 * 2, 2), :]
                for k in range(_K):
                    v = bufs[j, k, pl.ds(e0 * _K, _K)]
                    plsc.store_scatter(dst, [rowc, lanec[k]], v)

        def compact_write(t, j):
            transpose_into(j, _RCH)
            pltpu.async_copy(
                pks.at[j], out_hbm.at[pl.ds(pl.multiple_of(i0_of(t) // 8, _PK),
                                            _PK), :],
                osems.at[j])

        for j in range(_NBUF):
            fire(j, j)

        def step(i, carry):
            for j in range(_NBUF):
                wait_in(j)

                @pl.when(i > 0)
                def _():
                    wait_out(j)

                compact_write(_NBUF * i + j, j)
                fire(jnp.minimum(_NBUF * i + _NBUF + j, cpw - 1), j)
            return carry

        lax.fori_loop(0, cpw // _NBUF, step, 0)
        for j in range(_NBUF):
            wait_in(j)
            wait_out(j)

        if tail:
            @pl.when(wid == 0)
            def _():
                pltpu.sync_copy(tail_hbm, bufs.at[0, :, pl.ds(0, 128)])
                transpose_into(0, 128)
                pltpu.sync_copy(pks.at[0, pl.ds(0, 16), :],
                                out_hbm.at[pl.ds(npk - 16, 16), :])

    f = pl.kernel(
        body,
        out_type=jax.ShapeDtypeStruct((npk, 128), jnp.float32),
        mesh=mesh,
        compiler_params=pltpu.CompilerParams(needs_layout_passes=False),
        scratch_types=[
            pltpu.VMEM((_NBUF, _K, _RCH), jnp.float32),
            pltpu.VMEM((_NBUF, _PK, 128), jnp.float32),
            pltpu.SemaphoreType.DMA((_NBUF,)),
            pltpu.SemaphoreType.DMA((_NBUF,)),
        ],
    )
    return f(tT, tail_tT)


_CH = 16


def _sc_gather_fm(xT, tp):
    B = xT.shape[1]
    info = plsc.get_sparse_core_info()
    nc, ns = info.num_cores, info.num_subcores
    nw = nc * ns
    bw = B // nw
    gpf = bw // _CH
    mesh = plsc.VectorSubcoreMesh(core_axis_name="c", subcore_axis_name="s")

    def body(x_hbm, t_hbm, out_hbm, xv, tiles, rows, gsems, osems):
        wid = lax.axis_index("s") * nc + lax.axis_index("c")
        b0 = pl.multiple_of(wid * bw, bw)
        pltpu.sync_copy(x_hbm.at[:, pl.ds(b0, bw)], xv)
        iota = lax.iota(jnp.int32, _CH)

        def fire8(f, p):
            subs = []
            for c in range(gpf):
                raw = xv[f, pl.ds(c * _CH, _CH)]
                pltpu.async_copy(t_hbm.at[raw >> 3], tiles.at[p, c],
                                 gsems.at[p])
                subs.append((raw & 7) * _K)
            return tuple(subs)

        def drain8(p):
            for c in range(gpf):
                pltpu.make_async_copy(t_hbm.at[iota], tiles.at[p, c],
                                      gsems.at[p]).wait()

        def wait_out(p):
            pltpu.make_async_copy(rows.at[p], out_hbm.at[0, :, pl.ds(0, bw)],
                                  osems.at[p]).wait()

        def extract_write(f, p, subs):
            for c in range(gpf):
                for k in range(_K):
                    val = plsc.load_gather(tiles.at[p, c], [iota, subs[c] + k])
                    rows[p, k, pl.ds(c * _CH, _CH)] = val
            pltpu.async_copy(rows.at[p], out_hbm.at[f, :, pl.ds(b0, bw)],
                             osems.at[p])

        carry0 = fire8(0, 0) + fire8(1, 1)

        def step(i, carry):
            subsA, subsB = carry[:gpf], carry[gpf:]
            f0 = 2 * i
            new = []
            for p, subs, f in ((0, subsA, f0), (1, subsB, f0 + 1)):
                drain8(p)

                @pl.when(i > 0)
                def _():
                    wait_out(p)

                extract_write(f, p, subs)
                new.extend(fire8(jnp.minimum(f + 2, _FIELD - 1), p))
            return tuple(new)

        _ = lax.fori_loop(0, _FIELD // 2, step, carry0)
        for p in range(2):
            drain8(p)
            wait_out(p)

    f = pl.kernel(
        body,
        out_type=jax.ShapeDtypeStruct((_FIELD, _K, B), jnp.float32),
        mesh=mesh,
        compiler_params=pltpu.CompilerParams(needs_layout_passes=False),
        scratch_types=[
            pltpu.VMEM((_FIELD, bw), jnp.int32),
            pltpu.VMEM((2, gpf, _CH, 128), jnp.float32),
            pltpu.VMEM((2, _K, bw), jnp.float32),
            pltpu.SemaphoreType.DMA((2,)),
            pltpu.SemaphoreType.DMA((2,)),
        ],
    )
    return f(xT, tp)


def kernel(x, table, W, b, fi_rank):
    B, F = x.shape
    tT = table.T
    tp = _sc_repack(tT, tT[:, -128:])
    e3 = _sc_gather_fm(x.T, tp)
    outT = _tc_pairs(e3, W, b.reshape(_K, 1), fi_rank.reshape(_K, 1), 512)
    return outT.T
